# baseline (speedup 1.0000x reference)
import functools

import jax
import jax.numpy as jnp
from jax import lax
from jax.experimental import pallas as pl
from jax.experimental.pallas import tpu as pltpu
from jax.experimental.pallas import tpu_sc as plsc

NC = 2
NS = 16
NW = NC * NS
LANES = 16
CHUNK = 64
NB = 5
NI = 10
D = 128


def _make_edge_kernel(n_pad, ch_per_w):
    mesh = plsc.VectorSubcoreMesh(core_axis_name="c", subcore_axis_name="s")
    rpt = n_pad // NS

    @functools.partial(
        pl.kernel,
        out_type=jax.ShapeDtypeStruct((NC, n_pad, D), jnp.float32),
        mesh=mesh,
        scratch_types=[
            pltpu.VMEM((NI, 2, CHUNK), jnp.int32),
            pltpu.VMEM((NB, CHUNK, D), jnp.float32),
            pltpu.VMEM_SHARED((n_pad, D), jnp.float32),
        ] + [pltpu.SemaphoreType.DMA] * (NI + 2 * NB),
    )
    def edge_kernel(t_hbm, idx_hbm, out_hbm,
                    idx_v, rows_v, acc_s, *sems):
        isems = sems[0:NI]
        gsems = sems[NI:NI + NB]
        ssems = sems[NI + NB:NI + 2 * NB]
        cid = lax.axis_index("c")
        sid = lax.axis_index("s")
        wid = sid * NC + cid
        my_idx = idx_hbm.at[wid]
        ch = ch_per_w

        zeros = jnp.zeros((LANES,), jnp.float32)
        dv = D // LANES

        def zbody(i, carry):
            rows_v[0, i // dv, pl.ds((i % dv) * LANES, LANES)] = zeros
            return carry

        lax.fori_loop(0, CHUNK * dv, zbody, 0)
        for k in range(rpt // CHUNK):
            pltpu.sync_copy(
                rows_v.at[0],
                acc_s.at[pl.ds(sid * rpt + k * CHUNK, CHUNK)])
        plsc.subcore_barrier()

        pltpu.sync_copy(my_idx.at[0], idx_v.at[0])
        pltpu.async_copy(my_idx.at[1], idx_v.at[1], isems[1])
        pltpu.async_copy(my_idx.at[2], idx_v.at[2], isems[2])
        pltpu.async_copy(my_idx.at[3], idx_v.at[3], isems[3])
        pltpu.async_copy(t_hbm.at[idx_v.at[0, 0]], rows_v.at[0], gsems[0])
        pltpu.make_async_copy(my_idx.at[1], idx_v.at[1], isems[1]).wait()
        pltpu.async_copy(t_hbm.at[idx_v.at[1, 0]], rows_v.at[1], gsems[1])
        pltpu.make_async_copy(my_idx.at[2], idx_v.at[2], isems[2]).wait()
        pltpu.async_copy(t_hbm.at[idx_v.at[2, 0]], rows_v.at[2], gsems[2])

        def obody(jj, carry):
            for b in range(NI):
                j = jj * NI + b
                r0 = b % NB
                r3 = (b + 3) % NB
                q0 = b
                q3 = (b + 3) % NI
                q4 = (b + 4) % NI

                @pl.when(j >= 2)
                def _():
                    pltpu.make_async_copy(
                        rows_v.at[r3], acc_s.at[idx_v.at[(b + 8) % NI, 1]],
                        ssems[r3]).wait()

                @pl.when(j + 3 < ch)
                def _():
                    pltpu.make_async_copy(
                        my_idx.at[j + 3], idx_v.at[q3], isems[q3]).wait()
                    pltpu.async_copy(
                        t_hbm.at[idx_v.at[q3, 0]], rows_v.at[r3], gsems[r3])

                @pl.when(j + 4 < ch)
                def _():
                    pltpu.async_copy(my_idx.at[j + 4], idx_v.at[q4], isems[q4])

                pltpu.make_async_copy(
                    t_hbm.at[idx_v.at[q0, 0]], rows_v.at[r0], gsems[r0]).wait()
                pltpu.async_copy(rows_v.at[r0], acc_s.at[idx_v.at[q0, 1]],
                                 ssems[r0], add=True)
            return carry

        lax.fori_loop(0, ch // NI, obody, 0)
        for jt in (ch - 2, ch - 1):
            pltpu.make_async_copy(
                rows_v.at[jt % NB], acc_s.at[idx_v.at[jt % NI, 1]],
                ssems[jt % NB]).wait()
        plsc.subcore_barrier()
        pltpu.sync_copy(
            acc_s.at[pl.ds(sid * rpt, rpt)],
            out_hbm.at[cid, pl.ds(sid * rpt, rpt)])

    return edge_kernel


_BLK = 1024
_HB = 4096


def _tc_degree(idx2, n_bins):
    rows, hb = idx2.shape
    q_rows = n_bins // 128

    def body(i_ref, o_ref):
        step = pl.program_id(0)

        @pl.when(step == 0)
        def _():
            o_ref[...] = jnp.zeros_like(o_ref)

        acc = jnp.zeros((q_rows, D), jnp.float32)
        qi = lax.broadcasted_iota(jnp.int32, (q_rows, hb), 0)
        ri = lax.broadcasted_iota(jnp.int32, (D, hb), 0)
        for s in range(8):
            idxs = i_ref[s:s + 1, :]
            oh_q = (qi == (idxs >> 7)).astype(jnp.bfloat16)
            oh_r = (ri == (idxs & 127)).astype(jnp.bfloat16)
            acc += lax.dot_general(
                oh_q, oh_r, dimension_numbers=(((1,), (1,)), ((), ())),
                preferred_element_type=jnp.float32)
        o_ref[...] += acc

    return pl.pallas_call(
        body,
        grid=(rows // 8,),
        in_specs=[pl.BlockSpec((8, hb), lambda i: (i, 0))],
        out_specs=pl.BlockSpec((q_rows, D), lambda i: (0, 0)),
        out_shape=jax.ShapeDtypeStruct((q_rows, D), jnp.float32),
    )(idx2)


def _tc_matmul(x, w):
    n = x.shape[0]

    def body(x_ref, w_ref, o_ref):
        o_ref[...] = jnp.dot(x_ref[...], w_ref[...],
                             preferred_element_type=jnp.float32)

    return pl.pallas_call(
        body,
        grid=(n // _BLK,),
        in_specs=[pl.BlockSpec((_BLK, D), lambda i: (i, 0)),
                  pl.BlockSpec((D, D), lambda i: (0, 0))],
        out_specs=pl.BlockSpec((_BLK, D), lambda i: (i, 0)),
        out_shape=jax.ShapeDtypeStruct((n, D), jnp.float32),
    )(x, w)


def _tc_deg_scale(hist, xw):
    n_pad = xw.shape[0]

    def body(h_ref, xw_ref, dout_ref, din_ref, t_ref):
        rs = lax.rsqrt(jnp.maximum(h_ref[...], 1.0))
        dout_ref[...] = rs[0]
        din_ref[...] = rs[1]
        t_ref[...] = xw_ref[...] * rs[0][:, None]

    return pl.pallas_call(
        body,
        grid=(n_pad // _BLK,),
        in_specs=[pl.BlockSpec((2, _BLK), lambda i: (0, i)),
                  pl.BlockSpec((_BLK, D), lambda i: (i, 0))],
        out_specs=[pl.BlockSpec((_BLK,), lambda i: (i,)),
                   pl.BlockSpec((_BLK,), lambda i: (i,)),
                   pl.BlockSpec((_BLK, D), lambda i: (i, 0))],
        out_shape=[jax.ShapeDtypeStruct((n_pad,), jnp.float32),
                   jax.ShapeDtypeStruct((n_pad,), jnp.float32),
                   jax.ShapeDtypeStruct((n_pad, D), jnp.float32)],
    )(hist, xw)


def _tc_layer(acc, din, dout, b, w):
    n_pad = acc.shape[1]

    def body(a_ref, din_ref, dout_ref, b_ref, w_ref, o_ref):
        s = a_ref[0] + a_ref[1]
        h = s * din_ref[...][:, None] + b_ref[...][None, :]
        h = jnp.where(h > 0, h, 0.01 * h)
        o_ref[...] = jnp.dot(h, w_ref[...],
                             preferred_element_type=jnp.float32) \
            * dout_ref[...][:, None]

    return pl.pallas_call(
        body,
        grid=(n_pad // _BLK,),
        in_specs=[pl.BlockSpec((NC, _BLK, D), lambda i: (0, i, 0)),
                  pl.BlockSpec((_BLK,), lambda i: (i,)),
                  pl.BlockSpec((_BLK,), lambda i: (i,)),
                  pl.BlockSpec((D,), lambda i: (0,)),
                  pl.BlockSpec((D, D), lambda i: (0, 0))],
        out_specs=pl.BlockSpec((_BLK, D), lambda i: (i, 0)),
        out_shape=jax.ShapeDtypeStruct((n_pad, D), jnp.float32),
    )(acc, din, dout, b, w)


def _tc_final(acc, din, b):
    n_pad = acc.shape[1]

    def body(a_ref, din_ref, b_ref, o_ref):
        s = a_ref[0] + a_ref[1]
        h = s * din_ref[...][:, None] + b_ref[...][None, :]
        o_ref[...] = jnp.where(h > 0, h, 0.01 * h)

    return pl.pallas_call(
        body,
        grid=(n_pad // _BLK,),
        in_specs=[pl.BlockSpec((NC, _BLK, D), lambda i: (0, i, 0)),
                  pl.BlockSpec((_BLK,), lambda i: (i,)),
                  pl.BlockSpec((D,), lambda i: (0,))],
        out_specs=pl.BlockSpec((_BLK, D), lambda i: (i, 0)),
        out_shape=jax.ShapeDtypeStruct((n_pad, D), jnp.float32),
    )(acc, din, b)


def kernel(x, edge_index, W1, b1, W2, b2, W3, b3):
    n, d = x.shape
    e = edge_index.shape[1]
    assert d == D

    n_pad = -(-n // (NS * CHUNK)) * (NS * CHUNK)
    grp = NW * CHUNK * NI
    e_pad = -(-e // grp) * grp
    e_per_w = e_pad // NW
    ch_per_w = e_per_w // CHUNK

    src = edge_index[0].astype(jnp.int32)
    dst = edge_index[1].astype(jnp.int32)
    pad_ids = n + (jnp.arange(e_pad - e, dtype=jnp.int32) % (n_pad - n))
    src_p = jnp.concatenate([src, pad_ids])
    dst_p = jnp.concatenate([dst, pad_ids])
    idx_c = jnp.stack([src_p.reshape(NW, ch_per_w, CHUNK),
                       dst_p.reshape(NW, ch_per_w, CHUNK)], axis=2)

    x_p = jnp.pad(x, ((0, n_pad - n), (0, 0)))

    hist = _tc_degree(
        jnp.concatenate([src_p, dst_p + n_pad]).reshape(-1, _HB),
        2 * n_pad).reshape(2, n_pad)

    edge_kernel = _make_edge_kernel(n_pad, ch_per_w)

    xw = _tc_matmul(x_p, W1)
    dout, din, t = _tc_deg_scale(hist, xw)

    acc = edge_kernel(t, idx_c)
    t = _tc_layer(acc, din, dout, b1, W2)
    acc = edge_kernel(t, idx_c)
    t = _tc_layer(acc, din, dout, b2, W3)
    acc = edge_kernel(t, idx_c)
    out = _tc_final(acc, din, b3)
    return out[:n]


# speedup vs baseline: 1.1360x; 1.1360x over previous
import functools

import jax
import jax.numpy as jnp
from jax import lax
from jax.experimental import pallas as pl
from jax.experimental.pallas import tpu as pltpu
from jax.experimental.pallas import tpu_sc as plsc

NC = 2
NS = 16
NW = NC * NS
LANES = 16
CHUNK = 64
NB = 5
NI = 10
D = 128


def _make_edge_kernel(n_pad, ch_per_w):
    mesh = plsc.VectorSubcoreMesh(core_axis_name="c", subcore_axis_name="s")
    rpt = n_pad // NS

    @functools.partial(
        pl.kernel,
        out_type=jax.ShapeDtypeStruct((NC, n_pad, D), jnp.float32),
        mesh=mesh,
        scratch_types=[
            pltpu.VMEM((NI, 2, CHUNK), jnp.int32),
            pltpu.VMEM((NB, CHUNK, D), jnp.float32),
            pltpu.VMEM_SHARED((n_pad, D), jnp.float32),
        ] + [pltpu.SemaphoreType.DMA] * (NI + 2 * NB),
    )
    def edge_kernel(t_hbm, idx_hbm, out_hbm,
                    idx_v, rows_v, acc_s, *sems):
        isems = sems[0:NI]
        gsems = sems[NI:NI + NB]
        ssems = sems[NI + NB:NI + 2 * NB]
        cid = lax.axis_index("c")
        sid = lax.axis_index("s")
        wid = sid * NC + cid
        my_idx = idx_hbm.at[wid]
        ch = ch_per_w

        zeros = jnp.zeros((LANES,), jnp.float32)
        dv = D // LANES

        def zbody(i, carry):
            rows_v[0, i // dv, pl.ds((i % dv) * LANES, LANES)] = zeros
            return carry

        lax.fori_loop(0, CHUNK * dv, zbody, 0)
        for k in range(rpt // CHUNK):
            pltpu.sync_copy(
                rows_v.at[0],
                acc_s.at[pl.ds(sid * rpt + k * CHUNK, CHUNK)])
        plsc.subcore_barrier()

        pltpu.sync_copy(my_idx.at[0], idx_v.at[0])
        pltpu.async_copy(my_idx.at[1], idx_v.at[1], isems[1])
        pltpu.async_copy(my_idx.at[2], idx_v.at[2], isems[2])
        pltpu.async_copy(my_idx.at[3], idx_v.at[3], isems[3])
        pltpu.async_copy(t_hbm.at[idx_v.at[0, 0]], rows_v.at[0], gsems[0])
        pltpu.make_async_copy(my_idx.at[1], idx_v.at[1], isems[1]).wait()
        pltpu.async_copy(t_hbm.at[idx_v.at[1, 0]], rows_v.at[1], gsems[1])
        pltpu.make_async_copy(my_idx.at[2], idx_v.at[2], isems[2]).wait()
        pltpu.async_copy(t_hbm.at[idx_v.at[2, 0]], rows_v.at[2], gsems[2])

        def obody(jj, carry):
            for b in range(NI):
                j = jj * NI + b
                r0 = b % NB
                r3 = (b + 3) % NB
                q0 = b
                q3 = (b + 3) % NI
                q4 = (b + 4) % NI

                @pl.when(j >= 2)
                def _():
                    pltpu.make_async_copy(
                        rows_v.at[r3], acc_s.at[idx_v.at[(b + 8) % NI, 1]],
                        ssems[r3]).wait()

                @pl.when(j + 3 < ch)
                def _():
                    pltpu.make_async_copy(
                        my_idx.at[j + 3], idx_v.at[q3], isems[q3]).wait()
                    pltpu.async_copy(
                        t_hbm.at[idx_v.at[q3, 0]], rows_v.at[r3], gsems[r3])

                @pl.when(j + 4 < ch)
                def _():
                    pltpu.async_copy(my_idx.at[j + 4], idx_v.at[q4], isems[q4])

                pltpu.make_async_copy(
                    t_hbm.at[idx_v.at[q0, 0]], rows_v.at[r0], gsems[r0]).wait()
                pltpu.async_copy(rows_v.at[r0], acc_s.at[idx_v.at[q0, 1]],
                                 ssems[r0], add=True)
            return carry

        lax.fori_loop(0, ch // NI, obody, 0)
        for jt in (ch - 2, ch - 1):
            pltpu.make_async_copy(
                rows_v.at[jt % NB], acc_s.at[idx_v.at[jt % NI, 1]],
                ssems[jt % NB]).wait()
        plsc.subcore_barrier()
        pltpu.sync_copy(
            acc_s.at[pl.ds(sid * rpt, rpt)],
            out_hbm.at[cid, pl.ds(sid * rpt, rpt)])

    return edge_kernel


_HB = 20480


def _make_hist_kernel(per_w):
    mesh = plsc.VectorSubcoreMesh(core_axis_name="c", subcore_axis_name="s")

    @functools.partial(
        pl.kernel,
        out_type=jax.ShapeDtypeStruct((NW, _HB), jnp.float32),
        mesh=mesh,
        scratch_types=[
            pltpu.VMEM((per_w,), jnp.int32),
            pltpu.VMEM((_HB,), jnp.float32),
        ],
        compiler_params=pltpu.CompilerParams(needs_layout_passes=False),
    )
    def hist_kernel(idx_hbm, out_hbm, idx_v, hist_v):
        cid = lax.axis_index("c")
        sid = lax.axis_index("s")
        wid = sid * NC + cid
        zeros = jnp.zeros((LANES,), jnp.float32)
        ones = jnp.ones((LANES,), jnp.float32)

        def zbody(i, carry):
            hist_v[pl.ds(i * LANES, LANES)] = zeros
            return carry

        lax.fori_loop(0, _HB // LANES, zbody, 0)
        pltpu.sync_copy(idx_hbm.at[wid], idx_v)

        def hbody(i, carry):
            iv = idx_v[pl.ds(i * LANES, LANES)]
            plsc.addupdate_scatter(hist_v, [iv], ones)
            return carry

        lax.fori_loop(0, per_w // LANES, hbody, 0)
        pltpu.sync_copy(hist_v, out_hbm.at[wid])

    return hist_kernel


_BLK = 1024


def _tc_prelayer(hists, x, w):
    n_pad = x.shape[0]

    def body(h_ref, x_ref, w_ref, dout_ref, din_ref, t_ref):
        deg = jnp.sum(h_ref[...], axis=0)
        rs = lax.rsqrt(jnp.maximum(deg, 1.0))
        dout_ref[...] = rs[0]
        din_ref[...] = rs[1]
        t_ref[...] = jnp.dot(x_ref[...], w_ref[...],
                             preferred_element_type=jnp.float32) \
            * rs[0][:, None]

    return pl.pallas_call(
        body,
        grid=(n_pad // _BLK,),
        in_specs=[pl.BlockSpec((NW, 2, _BLK), lambda i: (0, 0, i)),
                  pl.BlockSpec((_BLK, D), lambda i: (i, 0)),
                  pl.BlockSpec((D, D), lambda i: (0, 0))],
        out_specs=[pl.BlockSpec((_BLK,), lambda i: (i,)),
                   pl.BlockSpec((_BLK,), lambda i: (i,)),
                   pl.BlockSpec((_BLK, D), lambda i: (i, 0))],
        out_shape=[jax.ShapeDtypeStruct((n_pad,), jnp.float32),
                   jax.ShapeDtypeStruct((n_pad,), jnp.float32),
                   jax.ShapeDtypeStruct((n_pad, D), jnp.float32)],
    )(hists, x, w)


def _tc_layer(acc, din, dout, b, w):
    n_pad = acc.shape[1]

    def body(a_ref, din_ref, dout_ref, b_ref, w_ref, o_ref):
        s = a_ref[0] + a_ref[1]
        h = s * din_ref[...][:, None] + b_ref[...][None, :]
        h = jnp.where(h > 0, h, 0.01 * h)
        o_ref[...] = jnp.dot(h, w_ref[...],
                             preferred_element_type=jnp.float32) \
            * dout_ref[...][:, None]

    return pl.pallas_call(
        body,
        grid=(n_pad // _BLK,),
        in_specs=[pl.BlockSpec((NC, _BLK, D), lambda i: (0, i, 0)),
                  pl.BlockSpec((_BLK,), lambda i: (i,)),
                  pl.BlockSpec((_BLK,), lambda i: (i,)),
                  pl.BlockSpec((D,), lambda i: (0,)),
                  pl.BlockSpec((D, D), lambda i: (0, 0))],
        out_specs=pl.BlockSpec((_BLK, D), lambda i: (i, 0)),
        out_shape=jax.ShapeDtypeStruct((n_pad, D), jnp.float32),
    )(acc, din, dout, b, w)


def _tc_final(acc, din, b):
    n_pad = acc.shape[1]

    def body(a_ref, din_ref, b_ref, o_ref):
        s = a_ref[0] + a_ref[1]
        h = s * din_ref[...][:, None] + b_ref[...][None, :]
        o_ref[...] = jnp.where(h > 0, h, 0.01 * h)

    return pl.pallas_call(
        body,
        grid=(n_pad // _BLK,),
        in_specs=[pl.BlockSpec((NC, _BLK, D), lambda i: (0, i, 0)),
                  pl.BlockSpec((_BLK,), lambda i: (i,)),
                  pl.BlockSpec((D,), lambda i: (0,))],
        out_specs=pl.BlockSpec((_BLK, D), lambda i: (i, 0)),
        out_shape=jax.ShapeDtypeStruct((n_pad, D), jnp.float32),
    )(acc, din, b)


def kernel(x, edge_index, W1, b1, W2, b2, W3, b3):
    n, d = x.shape
    e = edge_index.shape[1]
    assert d == D

    n_pad = -(-n // (NS * CHUNK)) * (NS * CHUNK)
    grp = NW * CHUNK * NI
    e_pad = -(-e // grp) * grp
    e_per_w = e_pad // NW
    ch_per_w = e_per_w // CHUNK

    src = edge_index[0].astype(jnp.int32)
    dst = edge_index[1].astype(jnp.int32)
    pad_ids = n + (jnp.arange(e_pad - e, dtype=jnp.int32) % (n_pad - n))
    src_p = jnp.concatenate([src, pad_ids])
    dst_p = jnp.concatenate([dst, pad_ids])
    idx_c = jnp.stack([src_p.reshape(NW, ch_per_w, CHUNK),
                       dst_p.reshape(NW, ch_per_w, CHUNK)], axis=2)

    x_p = jnp.pad(x, ((0, n_pad - n), (0, 0)))

    per_w = 2 * e_pad // NW
    hists = _make_hist_kernel(per_w)(
        jnp.concatenate([src_p, dst_p + n_pad]).reshape(NW, per_w))

    edge_kernel = _make_edge_kernel(n_pad, ch_per_w)

    dout, din, t = _tc_prelayer(hists.reshape(NW, 2, n_pad), x_p, W1)

    acc = edge_kernel(t, idx_c)
    t = _tc_layer(acc, din, dout, b1, W2)
    acc = edge_kernel(t, idx_c)
    t = _tc_layer(acc, din, dout, b2, W3)
    acc = edge_kernel(t, idx_c)
    out = _tc_final(acc, din, b3)
    return out[:n]
